# initial kernel scaffold (unmeasured)
import jax
import jax.numpy as jnp
from jax import lax
from jax.experimental import pallas as pl
from jax.experimental.pallas import tpu as pltpu


def kernel(
    x,
):
    def body(*refs):
        pass

    out_shape = jax.ShapeDtypeStruct(..., jnp.float32)
    return pl.pallas_call(body, out_shape=out_shape)(...)


# baseline (device time: 31219 ns/iter reference)
import jax
import jax.numpy as jnp
from jax import lax
from jax.experimental import pallas as pl
from jax.experimental.pallas import tpu as pltpu

N_DEV = 8


def kernel(x):
    m_per, n = x.shape
    M = N_DEV * m_per

    def body(x_ref, out_ref, gather_ref, send_sems, recv_sems):
        my = lax.axis_index("i")
        left = (my - 1) % N_DEV
        right = (my + 1) % N_DEV

        barrier_sem = pltpu.get_barrier_semaphore()
        for nbr in [left, right]:
            pl.semaphore_signal(
                barrier_sem, inc=1,
                device_id=(nbr,), device_id_type=pl.DeviceIdType.MESH,
            )
        pl.semaphore_wait(barrier_sem, 2)

        gather_ref[pl.ds(my * m_per, m_per), :] = x_ref[:, :]

        for h in range(N_DEV - 1):
            origin = (my - h) % N_DEV
            rdma = pltpu.make_async_remote_copy(
                src_ref=gather_ref.at[pl.ds(origin * m_per, m_per)],
                dst_ref=gather_ref.at[pl.ds(origin * m_per, m_per)],
                send_sem=send_sems.at[h],
                recv_sem=recv_sems.at[h],
                device_id=(right,),
                device_id_type=pl.DeviceIdType.MESH,
            )
            rdma.start()
            rdma.wait()

        v = gather_ref[:, :]
        idx = lax.broadcasted_iota(jnp.int32, (M, n), 0)
        k = 2
        while k <= M:
            j = k // 2
            while j >= 1:
                up = (idx & j) == 0
                p = jnp.where(
                    up, pltpu.roll(v, M - j, 0), pltpu.roll(v, j, 0)
                )
                dirmask = (idx & k) == 0
                take_min = up == dirmask
                v = jnp.where(
                    take_min, jnp.minimum(v, p), jnp.maximum(v, p)
                )
                j //= 2
            k *= 2

        gather_ref[:, :] = v
        out_ref[:, :] = gather_ref[pl.ds(my * m_per, m_per), :]

    return pl.pallas_call(
        body,
        out_shape=jax.ShapeDtypeStruct((m_per, n), x.dtype),
        in_specs=[pl.BlockSpec(memory_space=pltpu.VMEM)],
        out_specs=pl.BlockSpec(memory_space=pltpu.VMEM),
        scratch_shapes=[
            pltpu.VMEM((M, n), x.dtype),
            pltpu.SemaphoreType.DMA((N_DEV - 1,)),
            pltpu.SemaphoreType.DMA((N_DEV - 1,)),
        ],
        compiler_params=pltpu.CompilerParams(collective_id=0),
    )(x)


# device time: 17484 ns/iter; 1.7856x vs baseline; 1.7856x over previous
import jax
import jax.numpy as jnp
from jax import lax
from jax.experimental import pallas as pl
from jax.experimental.pallas import tpu as pltpu

N_DEV = 8
R = 3


def _cmpx(v, idx, j, dirmask):
    L = v.shape[0]
    up = (idx & j) == 0
    p = jnp.where(up, pltpu.roll(v, L - j, 0), pltpu.roll(v, j, 0))
    take_min = up == dirmask
    return jnp.where(take_min, jnp.minimum(v, p), jnp.maximum(v, p))


def kernel(x):
    m_per, n = x.shape
    M = N_DEV * m_per

    def body(x_ref, out_ref, gather_ref, send_sems, recv_sems):
        my = lax.axis_index("i")

        barrier_sem = pltpu.get_barrier_semaphore()
        for r in range(R):
            pl.semaphore_signal(
                barrier_sem, inc=1,
                device_id=(my ^ (1 << r),),
                device_id_type=pl.DeviceIdType.MESH,
            )
        pl.semaphore_wait(barrier_sem, R)

        v = x_ref[:, :]
        flip = (my & 1) == 1
        idx = lax.broadcasted_iota(jnp.int32, (m_per, n), 0)
        k = 2
        while k <= m_per:
            j = k // 2
            while j >= 1:
                v = _cmpx(v, idx, j, ((idx & k) == 0) ^ flip)
                j //= 2
            k *= 2
        gather_ref[pl.ds(my * m_per, m_per), :] = v

        for r in range(R):
            s = 1 << r
            rows = s * m_per
            partner = my ^ s
            bstart = (my - (my & (s - 1))) * m_per
            rdma = pltpu.make_async_remote_copy(
                src_ref=gather_ref.at[pl.ds(bstart, rows)],
                dst_ref=gather_ref.at[pl.ds(bstart, rows)],
                send_sem=send_sems.at[r],
                recv_sem=recv_sems.at[r],
                device_id=(partner,),
                device_id_type=pl.DeviceIdType.MESH,
            )
            rdma.start()
            rdma.wait()

            rows2 = 2 * rows
            b2start = (my - (my & (2 * s - 1))) * m_per
            d_asc = ((my >> (r + 1)) & 1) == 0
            if r < R - 1:
                w = gather_ref[pl.ds(b2start, rows2), :]
                widx = lax.broadcasted_iota(jnp.int32, (rows2, n), 0)
                j = rows
                while j >= 1:
                    w = _cmpx(w, widx, j, d_asc)
                    j //= 2
                gather_ref[pl.ds(b2start, rows2), :] = w
            else:
                w = gather_ref[:, :]
                widx = lax.broadcasted_iota(jnp.int32, (M, n), 0)
                j = rows
                while j >= m_per:
                    w = _cmpx(w, widx, j, True)
                    j //= 2
                gather_ref[:, :] = w
                sl = gather_ref[pl.ds(my * m_per, m_per), :]
                j = m_per // 2
                while j >= 1:
                    sl = _cmpx(sl, idx, j, True)
                    j //= 2
                out_ref[:, :] = sl

    return pl.pallas_call(
        body,
        out_shape=jax.ShapeDtypeStruct((m_per, n), x.dtype),
        in_specs=[pl.BlockSpec(memory_space=pltpu.VMEM)],
        out_specs=pl.BlockSpec(memory_space=pltpu.VMEM),
        scratch_shapes=[
            pltpu.VMEM((M, n), x.dtype),
            pltpu.SemaphoreType.DMA((R,)),
            pltpu.SemaphoreType.DMA((R,)),
        ],
        compiler_params=pltpu.CompilerParams(collective_id=0),
    )(x)
